# baseline (speedup 1.0000x reference)
import functools

import jax
import jax.numpy as jnp
from jax import lax
from jax.experimental import pallas as pl
from jax.experimental.pallas import tpu as pltpu
from jax.experimental.pallas import tpu_sc as plsc

NUM_CODES = 8192
DIM = 256
NUM_TOKENS = 8192
TM = 256
TN = 2048
N_TOK_TILES = NUM_TOKENS // TM
N_CODE_TILES = NUM_CODES // TN


_RG = TM // 8
_RH = 2
_RGH = _RG // _RH
_LCH = NUM_CODES // 128


def _argmin_body(x_ref, e_ref, out_ref, dbufa_ref, dbufb_ref, esq_ref,
                 xsqa_ref, xsqb_ref):
    s = pl.program_id(0)

    @pl.when(s == 0)
    def _():
        e0 = e_ref[...]
        esq_ref[...] = jnp.broadcast_to(
            jnp.sum(e0 * e0, axis=1)[None, :], (8, NUM_CODES))

    def step_body(wref, wxsq, rref, rxsq):
        x = x_ref[...]
        xsq = jnp.sum(x * x, axis=1, keepdims=True)
        dot2 = lax.dot_general(x * jnp.float32(-2.0), e_ref[...],
                               (((1,), (1,)), ((), ())),
                               preferred_element_type=jnp.float32)
        wref[...] = dot2.reshape(_RG, 8, NUM_CODES)
        wxsq[...] = xsq.reshape(_RG, 8, 1)

        dot3 = rref[...]
        xsq3 = rxsq[...]
        dist = (xsq3 + dot3) + esq_ref[...][None]
        m = jnp.min(dist, axis=2, keepdims=True)
        iota = lax.broadcasted_iota(
            jnp.int32, (8, NUM_CODES), 1).astype(jnp.float32)
        cand = jnp.where(dist == m, iota[None], jnp.float32(1e9))
        idx = jnp.min(cand, axis=2, keepdims=True)
        out_ref[0, :, :] = idx.reshape(TM, 1).astype(jnp.int32)

    @pl.when(s % 2 == 0)
    def _():
        step_body(dbufa_ref, xsqa_ref, dbufb_ref, xsqb_ref)

    @pl.when(s % 2 == 1)
    def _():
        step_body(dbufb_ref, xsqb_ref, dbufa_ref, xsqa_ref)


def _argmin_call(flat, emb, tile_ofs, n_tiles):
    out = pl.pallas_call(
        _argmin_body,
        grid=(n_tiles + 1,),
        in_specs=[
            pl.BlockSpec((TM, DIM),
                         lambda s: (tile_ofs + jnp.minimum(s, n_tiles - 1), 0)),
            pl.BlockSpec((NUM_CODES, DIM), lambda s: (0, 0)),
        ],
        out_specs=pl.BlockSpec((1, TM, 1),
                               lambda s: (jnp.maximum(s, 1) - 1, 0, 0)),
        out_shape=jax.ShapeDtypeStruct((n_tiles, TM, 1), jnp.int32),
        scratch_shapes=[
            pltpu.VMEM((_RG, 8, NUM_CODES), jnp.float32),
            pltpu.VMEM((_RG, 8, NUM_CODES), jnp.float32),
            pltpu.VMEM((8, NUM_CODES), jnp.float32),
            pltpu.VMEM((_RG, 8, 1), jnp.float32),
            pltpu.VMEM((_RG, 8, 1), jnp.float32),
        ],
    )(flat, emb)
    return out.reshape(n_tiles * TM)


_NC = 2
_NS = 16
_NW = _NC * _NS
_CHUNK = 128


@functools.cache
def _sc_gather(rows):
    rpw = rows // _NW

    def body(table_hbm, idx_hbm, out_hbm, idx_v, rows_v, sem):
        wid = lax.axis_index("s") * _NC + lax.axis_index("c")
        r0 = wid * rpw
        pltpu.sync_copy(idx_hbm.at[pl.ds(r0, rpw)], idx_v)
        cps = [
            pltpu.async_copy(table_hbm.at[idx_v.at[r]], rows_v.at[r], sem)
            for r in range(rpw)
        ]
        for cp in cps:
            cp.wait()
        pltpu.sync_copy(rows_v, out_hbm.at[pl.ds(r0, rpw)])

    return pl.kernel(
        body,
        mesh=plsc.VectorSubcoreMesh(core_axis_name="c", subcore_axis_name="s"),
        out_type=jax.ShapeDtypeStruct((rows, _CHUNK, DIM), jnp.float32),
        scratch_types=[
            pltpu.VMEM((rpw, _CHUNK), jnp.int32),
            pltpu.VMEM((rpw, _CHUNK, DIM), jnp.float32),
            pltpu.SemaphoreType.DMA,
        ],
    )


FT = 1024
N_FIN = NUM_TOKENS // FT


def _finalize_body(x_ref, qa_ref, qb_ref, qst_ref, loss_ref, acc_ref):
    i = pl.program_id(0)
    x = x_ref[...]
    q = jnp.where(i < N_FIN // 2, qa_ref[...], qb_ref[...])
    d = q - x
    qst_ref[...] = x + d
    s = jnp.sum(d * d)

    @pl.when(i == 0)
    def _():
        acc_ref[0, 0] = s

    @pl.when(i > 0)
    def _():
        acc_ref[0, 0] = acc_ref[0, 0] + s

    @pl.when(i == pl.num_programs(0) - 1)
    def _():
        m = acc_ref[0, 0] / jnp.float32(NUM_TOKENS * DIM)
        loss_ref[0, 0] = m + 0.25 * m


def _finalize_call(flat, qa, qb):
    h = N_FIN // 2
    return pl.pallas_call(
        _finalize_body,
        grid=(N_FIN,),
        in_specs=[
            pl.BlockSpec((FT, DIM), lambda i: (i, 0)),
            pl.BlockSpec((FT, DIM), lambda i: (jnp.minimum(i, h - 1), 0)),
            pl.BlockSpec((FT, DIM), lambda i: (jnp.maximum(i - h, 0), 0)),
        ],
        out_specs=[
            pl.BlockSpec((FT, DIM), lambda i: (i, 0)),
            pl.BlockSpec(memory_space=pltpu.SMEM),
        ],
        out_shape=[
            jax.ShapeDtypeStruct((NUM_TOKENS, DIM), jnp.float32),
            jax.ShapeDtypeStruct((1, 1), jnp.float32),
        ],
        scratch_shapes=[pltpu.SMEM((1, 1), jnp.float32)],
    )(flat, qa, qb)


def kernel(inputs, embedding):
    B, C, H, W = inputs.shape
    flat = jnp.transpose(inputs, (0, 2, 3, 1)).reshape(NUM_TOKENS, DIM)
    half_tiles = N_TOK_TILES // 2
    half_rows = (NUM_TOKENS // 2) // _CHUNK
    gather = _sc_gather(half_rows)
    idx_a = _argmin_call(flat, embedding, 0, half_tiles)
    q_a = gather(embedding, idx_a.reshape(half_rows, _CHUNK))
    idx_b = _argmin_call(flat, embedding, half_tiles, half_tiles)
    q_b = gather(embedding, idx_b.reshape(half_rows, _CHUNK))
    qst_flat, loss = _finalize_call(
        flat,
        q_a.reshape(NUM_TOKENS // 2, DIM),
        q_b.reshape(NUM_TOKENS // 2, DIM),
    )
    qst = jnp.transpose(qst_flat.reshape(B, H, W, C), (0, 3, 1, 2))
    idx = jnp.concatenate([idx_a, idx_b])
    return qst, loss[0, 0], idx.reshape(B, H, W)


# speedup vs baseline: 1.0622x; 1.0622x over previous
import functools

import jax
import jax.numpy as jnp
from jax import lax
from jax.experimental import pallas as pl
from jax.experimental.pallas import tpu as pltpu
from jax.experimental.pallas import tpu_sc as plsc

NUM_CODES = 8192
DIM = 256
NUM_TOKENS = 8192
TM = 256
TN = 2048
N_TOK_TILES = NUM_TOKENS // TM
N_CODE_TILES = NUM_CODES // TN


_RG = TM // 8
_RH = 2
_RGH = _RG // _RH
_LCH = NUM_CODES // 128


def _argmin_body(x_ref, e_ref, out_ref, dbufa_ref, dbufb_ref, esq_ref,
                 xsqa_ref, xsqb_ref):
    s = pl.program_id(0)

    @pl.when(s == 0)
    def _():
        e0 = e_ref[...]
        esq_ref[...] = jnp.broadcast_to(
            jnp.sum(e0 * e0, axis=1)[None, :], (8, NUM_CODES))

    def step_body(wref, wxsq, rref, rxsq):
        x = x_ref[...]
        xsq = jnp.sum(x * x, axis=1, keepdims=True)
        dot2 = lax.dot_general(x * jnp.float32(-2.0), e_ref[...],
                               (((1,), (1,)), ((), ())),
                               preferred_element_type=jnp.float32)
        wref[...] = dot2.reshape(_RG, 8, NUM_CODES)
        wxsq[...] = xsq.reshape(_RG, 8, 1)

        dot3 = rref[...]
        xsq3 = rxsq[...]
        dist = (xsq3 + dot3) + esq_ref[...][None]
        m = jnp.min(dist, axis=2, keepdims=True)
        iota = lax.broadcasted_iota(
            jnp.int32, (8, NUM_CODES), 1).astype(jnp.float32)
        cand = jnp.where(dist == m, iota[None], jnp.float32(1e9))
        idx = jnp.min(cand, axis=2, keepdims=True)
        out_ref[0, :, :] = idx.reshape(TM, 1).astype(jnp.int32)

    @pl.when(s % 2 == 0)
    def _():
        step_body(dbufa_ref, xsqa_ref, dbufb_ref, xsqb_ref)

    @pl.when(s % 2 == 1)
    def _():
        step_body(dbufb_ref, xsqb_ref, dbufa_ref, xsqa_ref)


def _argmin_call(flat, emb, tile_ofs, n_tiles):
    out = pl.pallas_call(
        _argmin_body,
        grid=(n_tiles + 1,),
        in_specs=[
            pl.BlockSpec((TM, DIM),
                         lambda s: (tile_ofs + jnp.minimum(s, n_tiles - 1), 0)),
            pl.BlockSpec((NUM_CODES, DIM), lambda s: (0, 0)),
        ],
        out_specs=pl.BlockSpec((1, TM, 1),
                               lambda s: (jnp.maximum(s, 1) - 1, 0, 0)),
        out_shape=jax.ShapeDtypeStruct((n_tiles, TM, 1), jnp.int32),
        scratch_shapes=[
            pltpu.VMEM((_RG, 8, NUM_CODES), jnp.float32),
            pltpu.VMEM((_RG, 8, NUM_CODES), jnp.float32),
            pltpu.VMEM((8, NUM_CODES), jnp.float32),
            pltpu.VMEM((_RG, 8, 1), jnp.float32),
            pltpu.VMEM((_RG, 8, 1), jnp.float32),
        ],
    )(flat, emb)
    return out.reshape(n_tiles * TM)


_NC = 2
_NS = 16
_NW = _NC * _NS
_CHUNK = 128


@functools.cache
def _sc_gather(rows):
    rpw = rows // _NW

    def body(table_hbm, idx_hbm, out_hbm, idx_v, rows_v, sem):
        wid = lax.axis_index("s") * _NC + lax.axis_index("c")
        r0 = wid * rpw
        pltpu.sync_copy(idx_hbm.at[pl.ds(r0, rpw)], idx_v)
        cps = [
            pltpu.async_copy(table_hbm.at[idx_v.at[r]], rows_v.at[r], sem)
            for r in range(rpw)
        ]
        for cp in cps:
            cp.wait()
        pltpu.sync_copy(rows_v, out_hbm.at[pl.ds(r0, rpw)])

    return pl.kernel(
        body,
        mesh=plsc.VectorSubcoreMesh(core_axis_name="c", subcore_axis_name="s"),
        out_type=jax.ShapeDtypeStruct((rows, _CHUNK, DIM), jnp.float32),
        scratch_types=[
            pltpu.VMEM((rpw, _CHUNK), jnp.int32),
            pltpu.VMEM((rpw, _CHUNK, DIM), jnp.float32),
            pltpu.SemaphoreType.DMA,
        ],
    )


FT = 1024
N_FIN = NUM_TOKENS // FT


def _finalize_body(x_ref, q_ref, qst_ref, loss_ref, acc_ref):
    i = pl.program_id(0)
    x = x_ref[...]
    q = q_ref[...]
    d = q - x
    qst_ref[...] = x + d
    s = jnp.sum(d * d)

    @pl.when(i == 0)
    def _():
        acc_ref[0, 0] = s

    @pl.when(i > 0)
    def _():
        acc_ref[0, 0] = acc_ref[0, 0] + s

    @pl.when(i == pl.num_programs(0) - 1)
    def _():
        m = acc_ref[0, 0] / jnp.float32(NUM_TOKENS * DIM)
        loss_ref[0, 0] = m + 0.25 * m


def _finalize_call(flat, q):
    return pl.pallas_call(
        _finalize_body,
        grid=(N_FIN,),
        in_specs=[
            pl.BlockSpec((FT, DIM), lambda i: (i, 0)),
            pl.BlockSpec((FT, DIM), lambda i: (i, 0)),
        ],
        out_specs=[
            pl.BlockSpec((FT, DIM), lambda i: (i, 0)),
            pl.BlockSpec(memory_space=pltpu.SMEM),
        ],
        out_shape=[
            jax.ShapeDtypeStruct((NUM_TOKENS, DIM), jnp.float32),
            jax.ShapeDtypeStruct((1, 1), jnp.float32),
        ],
        scratch_shapes=[pltpu.SMEM((1, 1), jnp.float32)],
    )(flat, q)


def kernel(inputs, embedding):
    B, C, H, W = inputs.shape
    flat = jnp.transpose(inputs, (0, 2, 3, 1)).reshape(NUM_TOKENS, DIM)
    rows = NUM_TOKENS // _CHUNK
    idx = _argmin_call(flat, embedding, 0, N_TOK_TILES)
    q = _sc_gather(rows)(embedding, idx.reshape(rows, _CHUNK))
    qst_flat, loss = _finalize_call(flat, q.reshape(NUM_TOKENS, DIM))
    qst = jnp.transpose(qst_flat.reshape(B, H, W, C), (0, 3, 1, 2))
    return qst, loss[0, 0], idx.reshape(B, H, W)
